# baseline (speedup 1.0000x reference)
import functools

import jax
import jax.numpy as jnp
from jax import lax
from jax.experimental import pallas as pl
from jax.experimental.pallas import tpu as pltpu
from jax.experimental.pallas import tpu_sc as plsc

N = 10000
E = 320000
D = 128
NC = 2
NS = 16
NW = NC * NS
EPT = E // NW
K = 40
NBUF = 6
CPB = 25
LAG = 5
NBODY = EPT // (K * CPB)
RPT = 624
TAIL = N - NS * RPT
ZR = 48

_MESH = plsc.VectorSubcoreMesh(core_axis_name="c", subcore_axis_name="s")


def _sc_degree(col3, z16):
    @functools.partial(
        pl.kernel,
        out_type=jax.ShapeDtypeStruct((NC, N, 16), jnp.float32),
        mesh=_MESH,
        scratch_types=[
            pltpu.VMEM((CPB, K), jnp.int32),
            pltpu.VMEM((K, 16), jnp.float32),
            pltpu.VMEM((ZR, 16), jnp.float32),
            pltpu.VMEM_SHARED((N, 16), jnp.float32),
        ] + [pltpu.SemaphoreType.DMA] * (2 * NBUF),
    )
    def k(col_hbm, z_hbm, out_hbm, colb, ones_v, zv, deg_sh, *dsem):
        cid = lax.axis_index("c")
        tid = lax.axis_index("s")
        wid = tid * NC + cid

        ones16 = jnp.full((16,), 1.0, jnp.float32)

        def fill_ones(i, _):
            ones_v[i, :] = ones16
            return 0

        lax.fori_loop(0, K, fill_ones, 0)

        rbase = pl.multiple_of(tid * RPT, 8)
        pltpu.sync_copy(z_hbm.at[pl.ds(0, ZR)], zv)
        for j in range(RPT // ZR):
            pltpu.sync_copy(zv, deg_sh.at[pl.ds(rbase + j * ZR, ZR)])

        @pl.when(tid == NS - 1)
        def _():
            pltpu.sync_copy(zv.at[pl.ds(0, TAIL)],
                            deg_sh.at[pl.ds(NS * RPT, TAIL)])

        plsc.subcore_barrier()

        def body(g, _):
            pltpu.sync_copy(col_hbm.at[wid, g], colb)
            scats = [None] * CPB
            ns = 2 * NBUF
            for j in range(CPB):
                if j >= ns:
                    scats[j - ns].wait()
                scats[j] = pltpu.async_copy(
                    ones_v, deg_sh.at[colb.at[j]], dsem[j % ns], add=True)
            for j in range(CPB - ns, CPB):
                scats[j].wait()
            return 0

        lax.fori_loop(0, NBODY, body, 0)
        plsc.subcore_barrier()
        pltpu.sync_copy(deg_sh.at[pl.ds(rbase, RPT)],
                        out_hbm.at[cid, pl.ds(rbase, RPT)])

        @pl.when(tid == NS - 1)
        def _():
            pltpu.sync_copy(deg_sh.at[pl.ds(NS * RPT, TAIL)],
                            out_hbm.at[cid, pl.ds(NS * RPT, TAIL)])

    return k(col3, z16)


def _sc_scatter(row3, col3, y, z128):
    @functools.partial(
        pl.kernel,
        out_type=jax.ShapeDtypeStruct((NC, N, D), jnp.float32),
        mesh=_MESH,
        scratch_types=[
            pltpu.VMEM((CPB, K), jnp.int32),
            pltpu.VMEM((CPB, K), jnp.int32),
            pltpu.VMEM((NBUF, K, D), jnp.float32),
            pltpu.VMEM((ZR, D), jnp.float32),
            pltpu.VMEM_SHARED((N, D), jnp.float32),
        ] + [pltpu.SemaphoreType.DMA] * (2 * NBUF),
    )
    def k(row_hbm, col_hbm, y_hbm, z_hbm, out_hbm, rowb, colb, gbuf, zv,
          agg_sh, *sems):
        gsem = sems[:NBUF]
        ssem = sems[NBUF:]
        cid = lax.axis_index("c")
        tid = lax.axis_index("s")
        wid = tid * NC + cid

        rbase = pl.multiple_of(tid * RPT, 8)
        pltpu.sync_copy(z_hbm.at[pl.ds(0, ZR)], zv)
        for j in range(RPT // ZR):
            pltpu.sync_copy(zv, agg_sh.at[pl.ds(rbase + j * ZR, ZR)])

        @pl.when(tid == NS - 1)
        def _():
            pltpu.sync_copy(zv.at[pl.ds(0, TAIL)],
                            agg_sh.at[pl.ds(NS * RPT, TAIL)])

        plsc.subcore_barrier()

        def body(g, _):
            pltpu.sync_copy(row_hbm.at[wid, g], rowb)
            pltpu.sync_copy(col_hbm.at[wid, g], colb)
            gathers = [None] * CPB
            scats = [None] * CPB

            def issue_gather(j):
                gathers[j] = pltpu.async_copy(
                    y_hbm.at[rowb.at[j]], gbuf.at[j % NBUF], gsem[j % NBUF])

            def issue_scatter(j):
                gathers[j].wait()
                scats[j] = pltpu.async_copy(
                    gbuf.at[j % NBUF], agg_sh.at[colb.at[j]],
                    ssem[j % NBUF], add=True)

            for j in range(CPB):
                if j >= NBUF:
                    scats[j - NBUF].wait()
                issue_gather(j)
                if j >= LAG:
                    issue_scatter(j - LAG)
            for j in range(CPB - LAG, CPB):
                issue_scatter(j)
            for j in range(CPB - NBUF, CPB):
                scats[j].wait()
            return 0

        lax.fori_loop(0, NBODY, body, 0)
        plsc.subcore_barrier()
        pltpu.sync_copy(agg_sh.at[pl.ds(rbase, RPT)],
                        out_hbm.at[cid, pl.ds(rbase, RPT)])

        @pl.when(tid == NS - 1)
        def _():
            pltpu.sync_copy(agg_sh.at[pl.ds(NS * RPT, TAIL)],
                            out_hbm.at[cid, pl.ds(NS * RPT, TAIL)])

    return k(row3, col3, y, z128)


def _dis_from(deg16):
    d = deg16[...]
    deg = d[0, :, 0:1] + d[1, :, 0:1] + 1.0
    return lax.rsqrt(deg)


def _tc_first(x, W1, deg16):
    def body(x_ref, w_ref, deg_ref, y_ref):
        dis = _dis_from(deg_ref)
        y_ref[...] = jnp.dot(x_ref[...], w_ref[...],
                             preferred_element_type=jnp.float32) * dis

    return pl.pallas_call(
        body, out_shape=jax.ShapeDtypeStruct((N, D), jnp.float32),
    )(x, W1, deg16)


def _tc_mid(y1, agg1, deg16, b1, gamma1, beta1, W2):
    def body(y_ref, a_ref, deg_ref, b_ref, g_ref, be_ref, w_ref, o_ref):
        dis = _dis_from(deg_ref)
        a = a_ref[...]
        h = dis * (a[0] + a[1] + y_ref[...]) + b_ref[...]
        mean = jnp.mean(h, axis=0, keepdims=True)
        var = jnp.mean((h - mean) ** 2, axis=0, keepdims=True)
        h = (h - mean) * lax.rsqrt(var + 1e-5) * g_ref[...] + be_ref[...]
        h = jnp.where(h > 0, h, jnp.exp(h) - 1.0)
        o_ref[...] = jnp.dot(h, w_ref[...],
                             preferred_element_type=jnp.float32) * dis

    return pl.pallas_call(
        body, out_shape=jax.ShapeDtypeStruct((N, D), jnp.float32),
    )(y1, agg1, deg16, b1.reshape(1, D), gamma1.reshape(1, D),
      beta1.reshape(1, D), W2)


def _tc_last(y2, agg2, deg16, b2):
    def body(y_ref, a_ref, deg_ref, b_ref, o_ref):
        dis = _dis_from(deg_ref)
        a = a_ref[...]
        o_ref[...] = dis * (a[0] + a[1] + y_ref[...]) + b_ref[...]

    return pl.pallas_call(
        body, out_shape=jax.ShapeDtypeStruct((N, D), jnp.float32),
    )(y2, agg2, deg16, b2.reshape(1, D))


def kernel(x, edge_index, W1, b1, gamma1, beta1, W2, b2):
    row3 = edge_index[0].astype(jnp.int32).reshape(NW, NBODY, CPB, K)
    col3 = edge_index[1].astype(jnp.int32).reshape(NW, NBODY, CPB, K)
    z16 = jnp.zeros((RPT, 16), jnp.float32)
    z128 = jnp.zeros((RPT, D), jnp.float32)
    deg16 = _sc_degree(col3, z16)
    y1 = _tc_first(x, W1, deg16)
    agg1 = _sc_scatter(row3, col3, y1, z128)
    y2 = _tc_mid(y1, agg1, deg16, b1, gamma1, beta1, W2)
    agg2 = _sc_scatter(row3, col3, y2, z128)
    return _tc_last(y2, agg2, deg16, b2)


# speedup vs baseline: 1.0855x; 1.0855x over previous
import functools

import jax
import jax.numpy as jnp
from jax import lax
from jax.experimental import pallas as pl
from jax.experimental.pallas import tpu as pltpu
from jax.experimental.pallas import tpu_sc as plsc

N = 10000
E = 320000
D = 128
NC = 2
NS = 16
NW = NC * NS
EPT = E // NW
K = 80
NBUF = 3
CPB = 25
LAG = 2
NBODY = EPT // (K * CPB)
RPT = 624
TAIL = N - NS * RPT
ZR = 48

_MESH = plsc.VectorSubcoreMesh(core_axis_name="c", subcore_axis_name="s")


def _sc_degree(col3, z16):
    @functools.partial(
        pl.kernel,
        out_type=jax.ShapeDtypeStruct((NC, N, 16), jnp.float32),
        mesh=_MESH,
        scratch_types=[
            pltpu.VMEM((CPB, K), jnp.int32),
            pltpu.VMEM((K, 16), jnp.float32),
            pltpu.VMEM((ZR, 16), jnp.float32),
            pltpu.VMEM_SHARED((N, 16), jnp.float32),
        ] + [pltpu.SemaphoreType.DMA] * (2 * NBUF),
    )
    def k(col_hbm, z_hbm, out_hbm, colb, ones_v, zv, deg_sh, *dsem):
        cid = lax.axis_index("c")
        tid = lax.axis_index("s")
        wid = tid * NC + cid

        ones16 = jnp.full((16,), 1.0, jnp.float32)

        def fill_ones(i, _):
            ones_v[i, :] = ones16
            return 0

        lax.fori_loop(0, K, fill_ones, 0)

        rbase = pl.multiple_of(tid * RPT, 8)
        pltpu.sync_copy(z_hbm.at[pl.ds(0, ZR)], zv)
        for j in range(RPT // ZR):
            pltpu.sync_copy(zv, deg_sh.at[pl.ds(rbase + j * ZR, ZR)])

        @pl.when(tid == NS - 1)
        def _():
            pltpu.sync_copy(zv.at[pl.ds(0, TAIL)],
                            deg_sh.at[pl.ds(NS * RPT, TAIL)])

        plsc.subcore_barrier()

        def body(g, _):
            pltpu.sync_copy(col_hbm.at[wid, g], colb)
            scats = [None] * CPB
            ns = 2 * NBUF
            for j in range(CPB):
                if j >= ns:
                    scats[j - ns].wait()
                scats[j] = pltpu.async_copy(
                    ones_v, deg_sh.at[colb.at[j]], dsem[j % ns], add=True)
            for j in range(CPB - ns, CPB):
                scats[j].wait()
            return 0

        lax.fori_loop(0, NBODY, body, 0)
        plsc.subcore_barrier()
        pltpu.sync_copy(deg_sh.at[pl.ds(rbase, RPT)],
                        out_hbm.at[cid, pl.ds(rbase, RPT)])

        @pl.when(tid == NS - 1)
        def _():
            pltpu.sync_copy(deg_sh.at[pl.ds(NS * RPT, TAIL)],
                            out_hbm.at[cid, pl.ds(NS * RPT, TAIL)])

    return k(col3, z16)


def _sc_scatter(row3, col3, y, z128):
    @functools.partial(
        pl.kernel,
        out_type=jax.ShapeDtypeStruct((NC, N, D), jnp.float32),
        mesh=_MESH,
        scratch_types=[
            pltpu.VMEM((CPB, K), jnp.int32),
            pltpu.VMEM((CPB, K), jnp.int32),
            pltpu.VMEM((NBUF, K, D), jnp.float32),
            pltpu.VMEM((ZR, D), jnp.float32),
            pltpu.VMEM_SHARED((N, D), jnp.float32),
        ] + [pltpu.SemaphoreType.DMA] * (2 * NBUF),
    )
    def k(row_hbm, col_hbm, y_hbm, z_hbm, out_hbm, rowb, colb, gbuf, zv,
          agg_sh, *sems):
        gsem = sems[:NBUF]
        ssem = sems[NBUF:]
        cid = lax.axis_index("c")
        tid = lax.axis_index("s")
        wid = tid * NC + cid

        rbase = pl.multiple_of(tid * RPT, 8)
        pltpu.sync_copy(z_hbm.at[pl.ds(0, ZR)], zv)
        for j in range(RPT // ZR):
            pltpu.sync_copy(zv, agg_sh.at[pl.ds(rbase + j * ZR, ZR)])

        @pl.when(tid == NS - 1)
        def _():
            pltpu.sync_copy(zv.at[pl.ds(0, TAIL)],
                            agg_sh.at[pl.ds(NS * RPT, TAIL)])

        plsc.subcore_barrier()

        def body(g, _):
            pltpu.sync_copy(row_hbm.at[wid, g], rowb)
            pltpu.sync_copy(col_hbm.at[wid, g], colb)
            gathers = [None] * CPB
            scats = [None] * CPB

            def issue_gather(j):
                gathers[j] = pltpu.async_copy(
                    y_hbm.at[rowb.at[j]], gbuf.at[j % NBUF], gsem[j % NBUF])

            def issue_scatter(j):
                gathers[j].wait()
                scats[j] = pltpu.async_copy(
                    gbuf.at[j % NBUF], agg_sh.at[colb.at[j]],
                    ssem[j % NBUF], add=True)

            for j in range(CPB):
                if j >= NBUF:
                    scats[j - NBUF].wait()
                issue_gather(j)
                if j >= LAG:
                    issue_scatter(j - LAG)
            for j in range(CPB - LAG, CPB):
                issue_scatter(j)
            for j in range(CPB - NBUF, CPB):
                scats[j].wait()
            return 0

        lax.fori_loop(0, NBODY, body, 0)
        plsc.subcore_barrier()
        pltpu.sync_copy(agg_sh.at[pl.ds(rbase, RPT)],
                        out_hbm.at[cid, pl.ds(rbase, RPT)])

        @pl.when(tid == NS - 1)
        def _():
            pltpu.sync_copy(agg_sh.at[pl.ds(NS * RPT, TAIL)],
                            out_hbm.at[cid, pl.ds(NS * RPT, TAIL)])

    return k(row3, col3, y, z128)


def _dis_from(deg16):
    d = deg16[...]
    deg = d[0, :, 0:1] + d[1, :, 0:1] + 1.0
    return lax.rsqrt(deg)


def _tc_first(x, W1, deg16):
    def body(x_ref, w_ref, deg_ref, y_ref):
        dis = _dis_from(deg_ref)
        y_ref[...] = jnp.dot(x_ref[...], w_ref[...],
                             preferred_element_type=jnp.float32) * dis

    return pl.pallas_call(
        body, out_shape=jax.ShapeDtypeStruct((N, D), jnp.float32),
    )(x, W1, deg16)


def _tc_mid(y1, agg1, deg16, b1, gamma1, beta1, W2):
    def body(y_ref, a_ref, deg_ref, b_ref, g_ref, be_ref, w_ref, o_ref):
        dis = _dis_from(deg_ref)
        a = a_ref[...]
        h = dis * (a[0] + a[1] + y_ref[...]) + b_ref[...]
        mean = jnp.mean(h, axis=0, keepdims=True)
        var = jnp.mean((h - mean) ** 2, axis=0, keepdims=True)
        h = (h - mean) * lax.rsqrt(var + 1e-5) * g_ref[...] + be_ref[...]
        h = jnp.where(h > 0, h, jnp.exp(h) - 1.0)
        o_ref[...] = jnp.dot(h, w_ref[...],
                             preferred_element_type=jnp.float32) * dis

    return pl.pallas_call(
        body, out_shape=jax.ShapeDtypeStruct((N, D), jnp.float32),
    )(y1, agg1, deg16, b1.reshape(1, D), gamma1.reshape(1, D),
      beta1.reshape(1, D), W2)


def _tc_last(y2, agg2, deg16, b2):
    def body(y_ref, a_ref, deg_ref, b_ref, o_ref):
        dis = _dis_from(deg_ref)
        a = a_ref[...]
        o_ref[...] = dis * (a[0] + a[1] + y_ref[...]) + b_ref[...]

    return pl.pallas_call(
        body, out_shape=jax.ShapeDtypeStruct((N, D), jnp.float32),
    )(y2, agg2, deg16, b2.reshape(1, D))


def kernel(x, edge_index, W1, b1, gamma1, beta1, W2, b2):
    row3 = edge_index[0].astype(jnp.int32).reshape(NW, NBODY, CPB, K)
    col3 = edge_index[1].astype(jnp.int32).reshape(NW, NBODY, CPB, K)
    z16 = jnp.zeros((RPT, 16), jnp.float32)
    z128 = jnp.zeros((RPT, D), jnp.float32)
    deg16 = _sc_degree(col3, z16)
    y1 = _tc_first(x, W1, deg16)
    agg1 = _sc_scatter(row3, col3, y1, z128)
    y2 = _tc_mid(y1, agg1, deg16, b1, gamma1, beta1, W2)
    agg2 = _sc_scatter(row3, col3, y2, z128)
    return _tc_last(y2, agg2, deg16, b2)
